# speedup vs baseline: 1.0006x; 1.0006x over previous
import jax
import jax.numpy as jnp
from jax.experimental import pallas as pl
from jax.experimental.pallas import tpu as pltpu

ND = 50000
NT = 50000
B = 512


def _gcn(x, ei, W, b, n):
    h = x @ W
    loop = jnp.arange(n, dtype=ei.dtype)
    src = jnp.concatenate([ei[0], loop])
    dst = jnp.concatenate([ei[1], loop])
    deg = jnp.zeros((n,), h.dtype).at[dst].add(1.0)
    dis = jnp.where(deg > 0, 1.0 / jnp.sqrt(deg), 0.0)
    norm = dis[src] * dis[dst]
    out = jnp.zeros((n, h.shape[1]), h.dtype).at[dst].add(h[src] * norm[:, None])
    return out + b


def _gep(x, batch, nb):
    s = jax.ops.segment_sum(x, batch, num_segments=nb)
    c = jax.ops.segment_sum(jnp.ones((x.shape[0],), x.dtype), batch, num_segments=nb)
    return s / jnp.maximum(c, 1.0)[:, None]


def _head_kernel(hd_ref, ht_ref,
                 mF1W_ref, mF1b_ref, mF2W_ref, mF2b_ref,
                 pF1W_ref, pF1b_ref, pF2W_ref, pF2b_ref,
                 c1W_ref, c1b_ref, c2W_ref, c2b_ref, c3W_ref, c3b_ref,
                 z_ref):
    hd = jax.nn.relu(hd_ref[...] @ mF1W_ref[...] + mF1b_ref[...])
    hd = hd @ mF2W_ref[...] + mF2b_ref[...]
    ht = jax.nn.relu(ht_ref[...] @ pF1W_ref[...] + pF1b_ref[...])
    ht = ht @ pF2W_ref[...] + pF2b_ref[...]
    xj = jnp.concatenate([hd, ht], axis=1)
    z = jax.nn.relu(xj @ c1W_ref[...] + c1b_ref[...])
    z = jax.nn.relu(z @ c2W_ref[...] + c2b_ref[...])
    z = z @ c3W_ref[...] + c3b_ref[...]
    z_ref[...] = z


def kernel(xd, xd_ei, xd_batch, xt, xt_ei, xt_batch, y,
           mW1, mB1, mW2, mB2, mW3, mB3, mF1W, mF1b, mF2W, mF2b,
           pW1, pB1, pW2, pB2, pW3, pB3, pF1W, pF1b, pF2W, pF2b,
           c1W, c1b, c2W, c2b, c3W, c3b):
    h = jax.nn.relu(_gcn(xd, xd_ei, mW1, mB1, ND))
    h = jax.nn.relu(_gcn(h, xd_ei, mW2, mB2, ND))
    h = jax.nn.relu(_gcn(h, xd_ei, mW3, mB3, ND))
    h = _gep(h, xd_batch, B)
    t = jax.nn.relu(_gcn(xt, xt_ei, pW1, pB1, NT))
    t = jax.nn.relu(_gcn(t, xt_ei, pW2, pB2, NT))
    t = jax.nn.relu(_gcn(t, xt_ei, pW3, pB3, NT))
    t = _gep(t, xt_batch, B)

    z2 = pl.pallas_call(
        _head_kernel,
        out_shape=jax.ShapeDtypeStruct((B, 1), jnp.float32),
    )(h, t, mF1W, mF1b, mF2W, mF2b, pF1W, pF1b, pF2W, pF2b,
      c1W, c1b, c2W, c2b, c3W, c3b)
    return (z2[:, 0], y)


# speedup vs baseline: 4.5973x; 4.5943x over previous
import functools

import jax
import jax.numpy as jnp
from jax import lax
from jax.experimental import pallas as pl
from jax.experimental.pallas import tpu as pltpu
from jax.experimental.pallas import tpu_sc as plsc

N = 50000
NP = 50176
NBLK = 392
NE = 800000
NEP = 819200
EROWS = 6400
EPW = 200
STRIPE = 3136
B = 512
BP = 640
JUNK = NP - 1

_MESH = plsc.VectorSubcoreMesh(core_axis_name="c", subcore_axis_name="s")
_SC_PARAMS = pltpu.CompilerParams(use_tc_tiling_on_sc=False)


def _zero_rows(buf, nrows, ncols):
    z = jnp.zeros((16,), jnp.float32)

    @pl.loop(0, nrows)
    def _(j):
        for k in range(ncols // 16):
            buf[j, pl.ds(k * 16, 16)] = z


def _deg_body(dst_hbm, out_hbm, dstc, ones, zbuf, acc, sem):
    c = lax.axis_index("c")
    t = lax.axis_index("s")
    w = c * 16 + t

    io = lax.iota(jnp.int32, 16)
    val = jnp.where(io == 0, 1.0, 0.0).astype(jnp.float32)

    @pl.loop(0, 128)
    def _(j):
        ones[j] = val

    _zero_rows(zbuf, 112, 16)
    for r in range(28):
        pltpu.sync_copy(zbuf, acc.at[pl.ds(t * STRIPE + r * 112, 112)])
    plsc.subcore_barrier()

    @pl.loop(0, EPW // 8)
    def _(blk):
        pltpu.async_copy(dst_hbm.at[pl.ds(w * EPW + blk * 8, 8)],
                         dstc, sem).wait()
        for j in range(8):
            pltpu.sync_copy(ones, acc.at[dstc.at[j]], add=True)

    plsc.subcore_barrier()
    pltpu.sync_copy(acc.at[pl.ds(t * STRIPE, STRIPE)],
                    out_hbm.at[c, pl.ds(t * STRIPE, STRIPE)])


def _make_deg():
    return pl.kernel(
        _deg_body,
        out_type=jax.ShapeDtypeStruct((2, NP, 16), jnp.float32),
        mesh=_MESH,
        compiler_params=_SC_PARAMS,
        scratch_types=[
            pltpu.VMEM((8, 128), jnp.int32),
            pltpu.VMEM((128, 16), jnp.float32),
            pltpu.VMEM((112, 16), jnp.float32),
            pltpu.VMEM_SHARED((NP, 16), jnp.float32),
            pltpu.SemaphoreType.DMA,
        ],
    )


def _agg_body(S, g_hbm, src_hbm, dst_hbm, out_hbm,
              srcc, dstc, adj, gbuf, zbuf, acc, sem):
    c = lax.axis_index("c")
    t = lax.axis_index("s")
    w = c * 16 + t

    _zero_rows(zbuf, 112, 32)
    for r in range(28):
        pltpu.sync_copy(zbuf, acc.at[pl.ds(t * STRIPE + r * 112, 112)])

    for s in range(S):
        plsc.subcore_barrier()

        @pl.loop(0, EPW // 8)
        def _(blk):
            pltpu.async_copy(src_hbm.at[pl.ds(w * EPW + blk * 8, 8)],
                             srcc, sem).wait()
            pltpu.async_copy(dst_hbm.at[pl.ds(w * EPW + blk * 8, 8)],
                             dstc, sem).wait()
            for j in range(8):
                for k in range(8):
                    adj[pl.ds(k * 16, 16)] = (
                        srcc[j, pl.ds(k * 16, 16)] * S + s)
                pltpu.async_copy(g_hbm.at[adj], gbuf, sem).wait()
                pltpu.sync_copy(gbuf, acc.at[dstc.at[j]], add=True)

        plsc.subcore_barrier()
        pltpu.sync_copy(
            acc.at[pl.ds(t * STRIPE, STRIPE)],
            out_hbm.at[c, s, pl.ds(t * STRIPE, STRIPE)])
        if s + 1 < S:
            for r in range(28):
                pltpu.sync_copy(zbuf,
                                acc.at[pl.ds(t * STRIPE + r * 112, 112)])


def _make_agg(S):
    return pl.kernel(
        functools.partial(_agg_body, S),
        out_type=jax.ShapeDtypeStruct((2, S, NP, 32), jnp.float32),
        mesh=_MESH,
        compiler_params=_SC_PARAMS,
        scratch_types=[
            pltpu.VMEM((8, 128), jnp.int32),
            pltpu.VMEM((8, 128), jnp.int32),
            pltpu.VMEM((128,), jnp.int32),
            pltpu.VMEM((128, 32), jnp.float32),
            pltpu.VMEM((112, 32), jnp.float32),
            pltpu.VMEM_SHARED((NP, 32), jnp.float32),
            pltpu.SemaphoreType.DMA,
        ],
    )


def _pool_body(S, h_hbm, b_hbm, out_hbm, bidx, pbuf, zbuf, acc, sem):
    c = lax.axis_index("c")
    t = lax.axis_index("s")
    w = c * 16 + t

    _zero_rows(zbuf, BP // 16, 32)
    pltpu.sync_copy(zbuf, acc.at[pl.ds(t * (BP // 16), BP // 16)])

    for s in range(S):
        plsc.subcore_barrier()

        @pl.loop(w, NBLK, step=32)
        def _(k):
            pltpu.async_copy(h_hbm.at[s, pl.ds(k * 128, 128)],
                             pbuf, sem).wait()
            pltpu.async_copy(b_hbm.at[k], bidx, sem).wait()
            pltpu.sync_copy(pbuf, acc.at[bidx], add=True)

        plsc.subcore_barrier()
        pltpu.sync_copy(acc.at[pl.ds(t * (BP // 16), BP // 16)],
                        out_hbm.at[c, s, pl.ds(t * (BP // 16), BP // 16)])
        if s + 1 < S:
            pltpu.sync_copy(zbuf, acc.at[pl.ds(t * (BP // 16), BP // 16)])


def _make_pool(S):
    return pl.kernel(
        functools.partial(_pool_body, S),
        out_type=jax.ShapeDtypeStruct((2, S, BP, 32), jnp.float32),
        mesh=_MESH,
        compiler_params=_SC_PARAMS,
        scratch_types=[
            pltpu.VMEM((128,), jnp.int32),
            pltpu.VMEM((128, 32), jnp.float32),
            pltpu.VMEM((BP // 16, 32), jnp.float32),
            pltpu.VMEM_SHARED((BP, 32), jnp.float32),
            pltpu.SemaphoreType.DMA,
        ],
    )


_R = 1024
_GRID = NP // _R


def _mm(a, b):
    return jnp.dot(a, b, precision=lax.Precision.HIGHEST,
                   preferred_element_type=jnp.float32)


def _dis_of(degP_ref):
    deg = degP_ref[0, :, 0] + degP_ref[1, :, 0] + 1.0
    return 1.0 / jnp.sqrt(deg)


def _lin1_body(x_ref, degP_ref, W_ref, o_ref):
    dis = _dis_of(degP_ref)
    o_ref[...] = _mm(x_ref[...], W_ref[...]) * dis[:, None]


def _cat_slices(P_ref):
    S = P_ref.shape[1]
    return jnp.concatenate(
        [P_ref[0, s] + P_ref[1, s] for s in range(S)], axis=-1)


def _lin_mid_body(P_ref, g_ref, degP_ref, b_ref, W_ref, o_ref):
    dis = _dis_of(degP_ref)
    hin = _cat_slices(P_ref) + g_ref[...]
    hin = jax.nn.relu(hin * dis[:, None] + b_ref[...])
    o_ref[...] = _mm(hin, W_ref[...]) * dis[:, None]


def _post_body(mark, P_ref, g_ref, degP_ref, b_ref, o_ref):
    dis = _dis_of(degP_ref)
    h = _cat_slices(P_ref) + g_ref[...]
    h = jax.nn.relu(h * dis[:, None] + b_ref[...])
    col = lax.broadcasted_iota(jnp.int32, h.shape, 1)
    h = jnp.where(col == mark, 1.0, h)
    for s in range(o_ref.shape[0]):
        o_ref[s] = h[:, s * 32:(s + 1) * 32]


def _tc_lin1(x, degP, W):
    Dp, Dn = W.shape
    return pl.pallas_call(
        _lin1_body,
        grid=(_GRID,),
        in_specs=[
            pl.BlockSpec((_R, Dp), lambda i: (i, 0)),
            pl.BlockSpec((2, _R, 16), lambda i: (0, i, 0)),
            pl.BlockSpec((Dp, Dn), lambda i: (0, 0)),
        ],
        out_specs=pl.BlockSpec((_R, Dn), lambda i: (i, 0)),
        out_shape=jax.ShapeDtypeStruct((NP, Dn), jnp.float32),
    )(x, degP, W)


def _tc_lin_mid(P, g, degP, b, W):
    Dp, Dn = W.shape
    S = Dp // 32
    return pl.pallas_call(
        _lin_mid_body,
        grid=(_GRID,),
        in_specs=[
            pl.BlockSpec((2, S, _R, 32), lambda i: (0, 0, i, 0)),
            pl.BlockSpec((_R, Dp), lambda i: (i, 0)),
            pl.BlockSpec((2, _R, 16), lambda i: (0, i, 0)),
            pl.BlockSpec((1, Dp), lambda i: (0, 0)),
            pl.BlockSpec((Dp, Dn), lambda i: (0, 0)),
        ],
        out_specs=pl.BlockSpec((_R, Dn), lambda i: (i, 0)),
        out_shape=jax.ShapeDtypeStruct((NP, Dn), jnp.float32),
    )(P, g, degP, b, W)


def _tc_post(P, g, degP, b, mark):
    Dp = g.shape[1]
    S = Dp // 32
    return pl.pallas_call(
        functools.partial(_post_body, mark),
        grid=(_GRID,),
        in_specs=[
            pl.BlockSpec((2, S, _R, 32), lambda i: (0, 0, i, 0)),
            pl.BlockSpec((_R, Dp), lambda i: (i, 0)),
            pl.BlockSpec((2, _R, 16), lambda i: (0, i, 0)),
            pl.BlockSpec((1, Dp), lambda i: (0, 0)),
        ],
        out_specs=pl.BlockSpec((S, _R, 32), lambda i: (0, i, 0)),
        out_shape=jax.ShapeDtypeStruct((S, NP, 32), jnp.float32),
    )(P, g, degP, b)


def _head_body(markd, markt,
               Qd_ref, Qt_ref,
               mF1W_ref, mF1b_ref, mF2W_ref, mF2b_ref,
               pF1W_ref, pF1b_ref, pF2W_ref, pF2b_ref,
               c1W_ref, c1b_ref, c2W_ref, c2b_ref, c3W_ref, c3b_ref,
               z_ref):
    qd = _cat_slices(Qd_ref)
    cd = jnp.maximum(qd[:B, markd], 1.0)
    pd = qd[:B, :] / cd[:, None]
    qt = _cat_slices(Qt_ref)
    ct = jnp.maximum(qt[:B, markt], 1.0)
    pt = qt[:B, :] / ct[:, None]

    hd = jax.nn.relu(_mm(pd, mF1W_ref[...]) + mF1b_ref[...])
    hd = _mm(hd, mF2W_ref[...]) + mF2b_ref[...]
    ht = jax.nn.relu(_mm(pt, pF1W_ref[...]) + pF1b_ref[...])
    ht = _mm(ht, pF2W_ref[...]) + pF2b_ref[...]
    xj = jnp.concatenate([hd, ht], axis=1)
    z = jax.nn.relu(_mm(xj, c1W_ref[...]) + c1b_ref[...])
    z = jax.nn.relu(_mm(z, c2W_ref[...]) + c2b_ref[...])
    z_ref[...] = _mm(z, c3W_ref[...]) + c3b_ref[...]


def _pad2(a, r, c):
    return jnp.pad(a, ((0, r - a.shape[0]), (0, c - a.shape[1])))


def _pad1(a, c):
    return jnp.pad(a, (0, c - a.shape[0])).reshape(1, c)


def _prep_edges(ei):
    src = jnp.pad(ei[0].astype(jnp.int32), (0, NEP - NE))
    dst = jnp.pad(ei[1].astype(jnp.int32), (0, NEP - NE),
                  constant_values=JUNK)
    return src.reshape(EROWS, 128), dst.reshape(EROWS, 128)


def _branch(x, ei, batch, W1, b1, W2, b2, W3, b3, dims, mark):
    D1, D2, D3 = dims
    src, dst = _prep_edges(ei)
    b2d = jnp.pad(batch.astype(jnp.int32), (0, NP - N),
                  constant_values=B).reshape(NBLK, 128)
    xp = _pad2(x, NP, W1.shape[0])

    degP = _make_deg()(dst)
    g1 = _tc_lin1(xp, degP, W1)
    P1 = _make_agg(D1 // 32)(g1.reshape(NP * (D1 // 32), 32), src, dst)
    g2 = _tc_lin_mid(P1, g1, degP, b1, W2)
    P2 = _make_agg(D2 // 32)(g2.reshape(NP * (D2 // 32), 32), src, dst)
    g3 = _tc_lin_mid(P2, g2, degP, b2, W3)
    P3 = _make_agg(D3 // 32)(g3.reshape(NP * (D3 // 32), 32), src, dst)
    h3 = _tc_post(P3, g3, degP, b3, mark)
    Q = _make_pool(D3 // 32)(h3, b2d)
    return Q


def kernel(xd, xd_ei, xd_batch, xt, xt_ei, xt_batch, y,
           mW1, mB1, mW2, mB2, mW3, mB3, mF1W, mF1b, mF2W, mF2b,
           pW1, pB1, pW2, pB2, pW3, pB3, pF1W, pF1b, pF2W, pF2b,
           c1W, c1b, c2W, c2b, c3W, c3b):
    Qd = _branch(xd, xd_ei, xd_batch,
                 _pad2(mW1, 64, 64), _pad1(mB1, 64),
                 _pad2(mW2, 64, 128), _pad1(mB2, 128),
                 _pad2(mW3, 128, 224), _pad1(mB3, 224),
                 (64, 128, 224), 220)
    Qt = _branch(xt, xt_ei, xt_batch,
                 _pad2(pW1, 64, 64), _pad1(pB1, 64),
                 _pad2(pW2, 64, 96), _pad1(pB2, 96),
                 _pad2(pW3, 96, 192), _pad1(pB3, 192),
                 (64, 96, 192), 164)

    z2 = pl.pallas_call(
        functools.partial(_head_body, 220, 164),
        out_shape=jax.ShapeDtypeStruct((B, 1), jnp.float32),
    )(Qd, Qt,
      _pad2(mF1W, 224, 1024), mF1b, mF2W, mF2b,
      _pad2(pF1W, 192, 1024), pF1b, pF2W, pF2b,
      c1W, c1b, c2W, c2b, c3W, c3b)
    return (z2[:, 0], y)


# speedup vs baseline: 5.9221x; 1.2882x over previous
import functools

import jax
import jax.numpy as jnp
from jax import lax
from jax.experimental import pallas as pl
from jax.experimental.pallas import tpu as pltpu
from jax.experimental.pallas import tpu_sc as plsc

N = 50000
NP = 50176
NBLK = 392
NE = 800000
NEP = 819200
EROWS = 6400
EPW = 200
STRIPE = 3136
B = 512
BP = 640
JUNK = NP - 1

_MESH = plsc.VectorSubcoreMesh(core_axis_name="c", subcore_axis_name="s")
_SC_PARAMS = pltpu.CompilerParams(use_tc_tiling_on_sc=False)


def _zero_rows(buf, nrows, ncols):
    z = jnp.zeros((16,), jnp.float32)

    @pl.loop(0, nrows)
    def _(j):
        for k in range(ncols // 16):
            buf[j, pl.ds(k * 16, 16)] = z


def _deg_body(ei_hbm, out_hbm, eic, ones, zbuf, acc, sem):
    c = lax.axis_index("c")
    t = lax.axis_index("s")
    w = c * 16 + t

    io = lax.iota(jnp.int32, 16)
    val = jnp.where(io == 0, 1.0, 0.0).astype(jnp.float32)

    @pl.loop(0, 128)
    def _(j):
        ones[j] = val

    _zero_rows(zbuf, 112, 16)
    for r in range(28):
        pltpu.sync_copy(zbuf, acc.at[pl.ds(t * STRIPE + r * 112, 112)])
    plsc.subcore_barrier()

    @pl.loop(0, EPW // 8)
    def _(blk):
        pltpu.async_copy(ei_hbm.at[pl.ds(w * EPW + blk * 8, 8)],
                         eic, sem).wait()
        for j in range(8):
            pltpu.sync_copy(ones, acc.at[eic.at[j, 1]], add=True)

    plsc.subcore_barrier()
    pltpu.sync_copy(acc.at[pl.ds(t * STRIPE, STRIPE)],
                    out_hbm.at[c, pl.ds(t * STRIPE, STRIPE)])


def _make_deg():
    return pl.kernel(
        _deg_body,
        out_type=jax.ShapeDtypeStruct((2, NP, 16), jnp.float32),
        mesh=_MESH,
        compiler_params=_SC_PARAMS,
        scratch_types=[
            pltpu.VMEM((8, 2, 128), jnp.int32),
            pltpu.VMEM((128, 16), jnp.float32),
            pltpu.VMEM((112, 16), jnp.float32),
            pltpu.VMEM_SHARED((NP, 16), jnp.float32),
            pltpu.SemaphoreType.DMA,
        ],
    )


def _agg_body(S, ei_hbm, g_hbm, z_hbm, out_hbm,
              eic, adj, gbuf, acc, isem, gsem):
    c = lax.axis_index("c")
    t = lax.axis_index("s")
    w = c * 16 + t

    pltpu.sync_copy(z_hbm, acc.at[pl.ds(t * STRIPE, STRIPE)])

    def idx_fetch(blk, half):
        return pltpu.async_copy(
            ei_hbm.at[pl.ds(w * EPW + blk * 8, 8)], eic.at[half], isem)

    def scat(j, h):
        p = (j - 3) % 4
        pltpu.make_async_copy(g_hbm.at[adj.at[p]], gbuf.at[p],
                              gsem.at[p]).wait()
        jj = (j - 3) % 8
        sel = h if j >= 3 else 1 - h
        pltpu.sync_copy(gbuf.at[p], acc.at[eic.at[sel, jj, 1]], add=True)

    def gat(s, j, h):
        q = j % 4
        for k in range(8):
            adj[q, pl.ds(k * 16, 16)] = eic[h, j, 0, pl.ds(k * 16, 16)] * S + s
        pltpu.async_copy(g_hbm.at[adj.at[q]], gbuf.at[q], gsem.at[q])

    for s in range(S):
        plsc.subcore_barrier()
        idx_fetch(0, 0).wait()

        @pl.loop(0, EPW // 8)
        def _(blk):
            h = blk % 2

            @pl.when(blk > 0)
            def _():
                pltpu.make_async_copy(
                    ei_hbm.at[pl.ds(w * EPW + blk * 8, 8)], eic.at[h],
                    isem).wait()

            for j in range(3):
                @pl.when(blk > 0)
                def _():
                    scat(j, h)
                gat(s, j, h)

            @pl.when(blk < (EPW // 8) - 1)
            def _():
                idx_fetch(blk + 1, 1 - h)

            for j in range(3, 8):
                scat(j, h)
                gat(s, j, h)

        for j in range(8, 11):
            scat(j, 0)

        plsc.subcore_barrier()
        pltpu.sync_copy(
            acc.at[pl.ds(t * STRIPE, STRIPE)],
            out_hbm.at[c, s, pl.ds(t * STRIPE, STRIPE)])
        if s + 1 < S:
            pltpu.sync_copy(z_hbm, acc.at[pl.ds(t * STRIPE, STRIPE)])


def _make_agg(S):
    return pl.kernel(
        functools.partial(_agg_body, S),
        out_type=jax.ShapeDtypeStruct((2, S, NP, 32), jnp.float32),
        mesh=_MESH,
        compiler_params=_SC_PARAMS,
        scratch_types=[
            pltpu.VMEM((2, 8, 2, 128), jnp.int32),
            pltpu.VMEM((4, 128), jnp.int32),
            pltpu.VMEM((4, 128, 32), jnp.float32),
            pltpu.VMEM_SHARED((NP, 32), jnp.float32),
            pltpu.SemaphoreType.DMA,
            pltpu.SemaphoreType.DMA((4,)),
        ],
    )


def _pool_body(S, h_hbm, b_hbm, out_hbm, bidx, pbuf, zbuf, acc, sem):
    c = lax.axis_index("c")
    t = lax.axis_index("s")
    w = c * 16 + t

    _zero_rows(zbuf, BP // 16, 32)
    pltpu.sync_copy(zbuf, acc.at[pl.ds(t * (BP // 16), BP // 16)])

    for s in range(S):
        plsc.subcore_barrier()

        @pl.loop(w, NBLK, step=32)
        def _(k):
            pltpu.async_copy(h_hbm.at[s, pl.ds(k * 128, 128)],
                             pbuf, sem).wait()
            pltpu.async_copy(b_hbm.at[k], bidx, sem).wait()
            pltpu.sync_copy(pbuf, acc.at[bidx], add=True)

        plsc.subcore_barrier()
        pltpu.sync_copy(acc.at[pl.ds(t * (BP // 16), BP // 16)],
                        out_hbm.at[c, s, pl.ds(t * (BP // 16), BP // 16)])
        if s + 1 < S:
            pltpu.sync_copy(zbuf, acc.at[pl.ds(t * (BP // 16), BP // 16)])


def _make_pool(S):
    return pl.kernel(
        functools.partial(_pool_body, S),
        out_type=jax.ShapeDtypeStruct((2, S, BP, 32), jnp.float32),
        mesh=_MESH,
        compiler_params=_SC_PARAMS,
        scratch_types=[
            pltpu.VMEM((128,), jnp.int32),
            pltpu.VMEM((128, 32), jnp.float32),
            pltpu.VMEM((BP // 16, 32), jnp.float32),
            pltpu.VMEM_SHARED((BP, 32), jnp.float32),
            pltpu.SemaphoreType.DMA,
        ],
    )


_R = 1024
_GRID = NP // _R


def _mm(a, b):
    return jnp.dot(a, b, precision=lax.Precision.HIGHEST,
                   preferred_element_type=jnp.float32)


def _dis_of(degP_ref):
    deg = degP_ref[0, :, 0] + degP_ref[1, :, 0] + 1.0
    return 1.0 / jnp.sqrt(deg)


def _lin1_body(x_ref, degP_ref, W_ref, o_ref):
    dis = _dis_of(degP_ref)
    o_ref[...] = _mm(x_ref[...], W_ref[...]) * dis[:, None]


def _cat_slices(P_ref):
    S = P_ref.shape[1]
    return jnp.concatenate(
        [P_ref[0, s] + P_ref[1, s] for s in range(S)], axis=-1)


def _lin_mid_body(P_ref, g_ref, degP_ref, b_ref, W_ref, o_ref):
    dis = _dis_of(degP_ref)
    hin = _cat_slices(P_ref) + g_ref[...]
    hin = jax.nn.relu(hin * dis[:, None] + b_ref[...])
    o_ref[...] = _mm(hin, W_ref[...]) * dis[:, None]


def _post_body(mark, P_ref, g_ref, degP_ref, b_ref, o_ref):
    dis = _dis_of(degP_ref)
    h = _cat_slices(P_ref) + g_ref[...]
    h = jax.nn.relu(h * dis[:, None] + b_ref[...])
    col = lax.broadcasted_iota(jnp.int32, h.shape, 1)
    h = jnp.where(col == mark, 1.0, h)
    for s in range(o_ref.shape[0]):
        o_ref[s] = h[:, s * 32:(s + 1) * 32]


def _tc_lin1(x, degP, W):
    Dp, Dn = W.shape
    return pl.pallas_call(
        _lin1_body,
        grid=(_GRID,),
        in_specs=[
            pl.BlockSpec((_R, Dp), lambda i: (i, 0)),
            pl.BlockSpec((2, _R, 16), lambda i: (0, i, 0)),
            pl.BlockSpec((Dp, Dn), lambda i: (0, 0)),
        ],
        out_specs=pl.BlockSpec((_R, Dn), lambda i: (i, 0)),
        out_shape=jax.ShapeDtypeStruct((NP, Dn), jnp.float32),
    )(x, degP, W)


def _tc_lin_mid(P, g, degP, b, W):
    Dp, Dn = W.shape
    S = Dp // 32
    return pl.pallas_call(
        _lin_mid_body,
        grid=(_GRID,),
        in_specs=[
            pl.BlockSpec((2, S, _R, 32), lambda i: (0, 0, i, 0)),
            pl.BlockSpec((_R, Dp), lambda i: (i, 0)),
            pl.BlockSpec((2, _R, 16), lambda i: (0, i, 0)),
            pl.BlockSpec((1, Dp), lambda i: (0, 0)),
            pl.BlockSpec((Dp, Dn), lambda i: (0, 0)),
        ],
        out_specs=pl.BlockSpec((_R, Dn), lambda i: (i, 0)),
        out_shape=jax.ShapeDtypeStruct((NP, Dn), jnp.float32),
    )(P, g, degP, b, W)


def _tc_post(P, g, degP, b, mark):
    Dp = g.shape[1]
    S = Dp // 32
    return pl.pallas_call(
        functools.partial(_post_body, mark),
        grid=(_GRID,),
        in_specs=[
            pl.BlockSpec((2, S, _R, 32), lambda i: (0, 0, i, 0)),
            pl.BlockSpec((_R, Dp), lambda i: (i, 0)),
            pl.BlockSpec((2, _R, 16), lambda i: (0, i, 0)),
            pl.BlockSpec((1, Dp), lambda i: (0, 0)),
        ],
        out_specs=pl.BlockSpec((S, _R, 32), lambda i: (0, i, 0)),
        out_shape=jax.ShapeDtypeStruct((S, NP, 32), jnp.float32),
    )(P, g, degP, b)


def _head_body(markd, markt,
               Qd_ref, Qt_ref,
               mF1W_ref, mF1b_ref, mF2W_ref, mF2b_ref,
               pF1W_ref, pF1b_ref, pF2W_ref, pF2b_ref,
               c1W_ref, c1b_ref, c2W_ref, c2b_ref, c3W_ref, c3b_ref,
               z_ref):
    qd = _cat_slices(Qd_ref)
    cd = jnp.maximum(qd[:B, markd], 1.0)
    pd = qd[:B, :] / cd[:, None]
    qt = _cat_slices(Qt_ref)
    ct = jnp.maximum(qt[:B, markt], 1.0)
    pt = qt[:B, :] / ct[:, None]

    hd = jax.nn.relu(_mm(pd, mF1W_ref[...]) + mF1b_ref[...])
    hd = _mm(hd, mF2W_ref[...]) + mF2b_ref[...]
    ht = jax.nn.relu(_mm(pt, pF1W_ref[...]) + pF1b_ref[...])
    ht = _mm(ht, pF2W_ref[...]) + pF2b_ref[...]
    xj = jnp.concatenate([hd, ht], axis=1)
    z = jax.nn.relu(_mm(xj, c1W_ref[...]) + c1b_ref[...])
    z = jax.nn.relu(_mm(z, c2W_ref[...]) + c2b_ref[...])
    z_ref[...] = _mm(z, c3W_ref[...]) + c3b_ref[...]


def _pad2(a, r, c):
    return jnp.pad(a, ((0, r - a.shape[0]), (0, c - a.shape[1])))


def _pad1(a, c):
    return jnp.pad(a, (0, c - a.shape[0])).reshape(1, c)


def _prep_edges(ei):
    src = jnp.pad(ei[0].astype(jnp.int32), (0, NEP - NE))
    dst = jnp.pad(ei[1].astype(jnp.int32), (0, NEP - NE),
                  constant_values=JUNK)
    return jnp.stack([src.reshape(EROWS, 128),
                      dst.reshape(EROWS, 128)], axis=1)


def _branch(x, ei, batch, W1, b1, W2, b2, W3, b3, dims, mark):
    D1, D2, D3 = dims
    eip = _prep_edges(ei)
    zrows = jnp.zeros((STRIPE, 32), jnp.float32)
    b2d = jnp.pad(batch.astype(jnp.int32), (0, NP - N),
                  constant_values=B).reshape(NBLK, 128)
    xp = _pad2(x, NP, W1.shape[0])

    degP = _make_deg()(eip)
    g1 = _tc_lin1(xp, degP, W1)
    P1 = _make_agg(D1 // 32)(eip, g1.reshape(NP * (D1 // 32), 32), zrows)
    g2 = _tc_lin_mid(P1, g1, degP, b1, W2)
    P2 = _make_agg(D2 // 32)(eip, g2.reshape(NP * (D2 // 32), 32), zrows)
    g3 = _tc_lin_mid(P2, g2, degP, b2, W3)
    P3 = _make_agg(D3 // 32)(eip, g3.reshape(NP * (D3 // 32), 32), zrows)
    h3 = _tc_post(P3, g3, degP, b3, mark)
    Q = _make_pool(D3 // 32)(h3, b2d)
    return Q


def kernel(xd, xd_ei, xd_batch, xt, xt_ei, xt_batch, y,
           mW1, mB1, mW2, mB2, mW3, mB3, mF1W, mF1b, mF2W, mF2b,
           pW1, pB1, pW2, pB2, pW3, pB3, pF1W, pF1b, pF2W, pF2b,
           c1W, c1b, c2W, c2b, c3W, c3b):
    Qd = _branch(xd, xd_ei, xd_batch,
                 _pad2(mW1, 64, 64), _pad1(mB1, 64),
                 _pad2(mW2, 64, 128), _pad1(mB2, 128),
                 _pad2(mW3, 128, 224), _pad1(mB3, 224),
                 (64, 128, 224), 220)
    Qt = _branch(xt, xt_ei, xt_batch,
                 _pad2(pW1, 64, 64), _pad1(pB1, 64),
                 _pad2(pW2, 64, 96), _pad1(pB2, 96),
                 _pad2(pW3, 96, 192), _pad1(pB3, 192),
                 (64, 96, 192), 164)

    z2 = pl.pallas_call(
        functools.partial(_head_body, 220, 164),
        out_shape=jax.ShapeDtypeStruct((B, 1), jnp.float32),
    )(Qd, Qt,
      _pad2(mF1W, 224, 1024), mF1b, mF2W, mF2b,
      _pad2(pF1W, 192, 1024), pF1b, pF2W, pF2b,
      c1W, c1b, c2W, c2b, c3W, c3b)
    return (z2[:, 0], y)
